# baseline (speedup 1.0000x reference)
import jax
import jax.numpy as jnp
from jax import lax
from jax.experimental import pallas as pl
from jax.experimental.pallas import tpu as pltpu
from jax.experimental.pallas import tpu_sc as plsc

BATCH = 4096
HIST = 200
EMBED_DIM = 16
N = BATCH * HIST
NUM_CORES = 2
NUM_SUBCORES = 16
NW = NUM_CORES * NUM_SUBCORES
PER_W = N // NW
CHUNK = 1600
NCHUNK = PER_W // CHUNK


def _body(tab_hbm, idx_hbm, out_hbm, idx_v, rows_v, sem):
    wid = lax.axis_index("s") * NUM_CORES + lax.axis_index("c")
    base = wid * PER_W

    def step(j, carry):
        off = base + j * CHUNK
        pltpu.sync_copy(idx_hbm.at[pl.ds(off, CHUNK)], idx_v)
        pltpu.async_copy(tab_hbm.at[idx_v], rows_v, sem).wait()
        pltpu.sync_copy(rows_v, out_hbm.at[pl.ds(off, CHUNK)])
        return carry

    lax.fori_loop(0, NCHUNK, step, 0)


def kernel(embeddings, x):
    xf = x.reshape(-1).astype(jnp.int32)
    mesh = plsc.VectorSubcoreMesh(core_axis_name="c", subcore_axis_name="s")
    f = pl.kernel(
        _body,
        mesh=mesh,
        compiler_params=pltpu.CompilerParams(use_tc_tiling_on_sc=False),
        out_type=jax.ShapeDtypeStruct((N, EMBED_DIM), jnp.float32),
        scratch_types=[
            pltpu.VMEM((CHUNK,), jnp.int32),
            pltpu.VMEM((CHUNK, EMBED_DIM), jnp.float32),
            pltpu.SemaphoreType.DMA,
        ],
    )
    out = f(embeddings, xf)
    return out.reshape(BATCH, HIST, EMBED_DIM)


# speedup vs baseline: 1.0209x; 1.0209x over previous
import jax
import jax.numpy as jnp
from jax import lax
from jax.experimental import pallas as pl
from jax.experimental.pallas import tpu as pltpu
from jax.experimental.pallas import tpu_sc as plsc

BATCH = 4096
HIST = 200
EMBED_DIM = 16
N = BATCH * HIST
NUM_CORES = 2
NUM_SUBCORES = 16
NW = NUM_CORES * NUM_SUBCORES
PER_W = N // NW
CHUNK = 1600
NCHUNK = PER_W // CHUNK
NBUF = 4


def _body(tab_hbm, idx_hbm, out_hbm, idx_v, rows, sem_g, sem_w):
    wid = lax.axis_index("s") * NUM_CORES + lax.axis_index("c")
    base = wid * PER_W

    pltpu.sync_copy(idx_hbm.at[pl.ds(base, PER_W)], idx_v)

    def gather(j):
        b = j % NBUF
        return pltpu.async_copy(
            tab_hbm.at[idx_v.at[pl.ds(j * CHUNK, CHUNK)]], rows[b], sem_g[b])

    def write(j):
        b = j % NBUF
        return pltpu.async_copy(
            rows[b], out_hbm.at[pl.ds(base + j * CHUNK, CHUNK)], sem_w[b])

    g_desc = [None] * NCHUNK
    w_desc = [None] * NCHUNK
    g_desc[0] = gather(0)
    g_desc[1] = gather(1)
    for j in range(NCHUNK):
        g_desc[j].wait()
        w_desc[j] = write(j)
        nxt = j + 2
        if nxt < NCHUNK:
            if nxt >= NBUF:
                w_desc[nxt - NBUF].wait()
            g_desc[nxt] = gather(nxt)
    w_desc[NCHUNK - 2].wait()
    w_desc[NCHUNK - 1].wait()


def kernel(embeddings, x):
    xf = x.reshape(-1).astype(jnp.int32)
    mesh = plsc.VectorSubcoreMesh(core_axis_name="c", subcore_axis_name="s")
    f = pl.kernel(
        _body,
        mesh=mesh,
        compiler_params=pltpu.CompilerParams(use_tc_tiling_on_sc=False),
        out_type=jax.ShapeDtypeStruct((N, EMBED_DIM), jnp.float32),
        scratch_types=[
            pltpu.VMEM((PER_W,), jnp.int32),
            [pltpu.VMEM((CHUNK, EMBED_DIM), jnp.float32) for _ in range(NBUF)],
            [pltpu.SemaphoreType.DMA for _ in range(NBUF)],
            [pltpu.SemaphoreType.DMA for _ in range(NBUF)],
        ],
    )
    out = f(embeddings, xf)
    return out.reshape(BATCH, HIST, EMBED_DIM)


# speedup vs baseline: 1.3672x; 1.3392x over previous
import jax
import jax.numpy as jnp
from jax import lax
from jax.experimental import pallas as pl
from jax.experimental.pallas import tpu as pltpu
from jax.experimental.pallas import tpu_sc as plsc

BATCH = 4096
HIST = 200
EMBED_DIM = 16
N = BATCH * HIST
NUM_CORES = 2
NUM_SUBCORES = 16
NW = NUM_CORES * NUM_SUBCORES
UNIT = 1024
NUNIT = N // UNIT
PER_W = NUNIT // NW


def _body(tab_hbm, idx_hbm, out_hbm, idx_vs, g_bufs, w_bufs,
          sem_i, sem_g, sem_w):
    wid = lax.axis_index("s") * NUM_CORES + lax.axis_index("c")
    lane = lax.iota(jnp.int32, 16)
    pattern = (lane >> 3) * 8192 + (lane & 7) * 128

    def unit_off(t):
        u = wid + t * NW
        h = u >> 2
        q = u & 3
        return h * BATCH + q * UNIT, h * 65536 + q * 8192

    def idx_copy(t):
        src_off, _ = unit_off(t)
        return pltpu.make_async_copy(
            idx_hbm.at[pl.ds(src_off, UNIT)], idx_vs.at[t & 1], sem_i)

    def gather_copy(t):
        par = t & 1
        return pltpu.make_async_copy(
            tab_hbm.at[idx_vs.at[par]], g_bufs.at[par], sem_g)

    def write_copy(t, p):
        _, dst_off = unit_off(t)
        return pltpu.make_async_copy(
            w_bufs.at[t & 1, pl.ds(p * 8192, 8192)],
            out_hbm.at[pl.ds(dst_off + p * 32768, 8192)], sem_w)

    idx_copy(0).start()
    idx_copy(0).wait()
    gather_copy(0).start()
    idx_copy(1).start()

    def step(t, carry):
        par = t & 1
        gather_copy(t).wait()

        @pl.when(t + 1 < PER_W)
        def _():
            idx_copy(t + 1).wait()
            gather_copy(t + 1).start()

        @pl.when(t + 2 < PER_W)
        def _():
            idx_copy(t + 2).start()

        @pl.when(t >= 2)
        def _():
            write_copy(t - 2, 0).wait()
            write_copy(t - 2, 1).wait()

        def trans(i, c2):
            row = g_bufs[par, i]
            base = ((i >> 7) << 10) + (i & 127)
            plsc.store_scatter(w_bufs.at[par], [pattern + base], row)
            return c2

        lax.fori_loop(0, UNIT, trans, 0)
        write_copy(t, 0).start()
        write_copy(t, 1).start()
        return carry

    lax.fori_loop(0, PER_W, step, 0)
    write_copy(PER_W - 2, 0).wait()
    write_copy(PER_W - 2, 1).wait()
    write_copy(PER_W - 1, 0).wait()
    write_copy(PER_W - 1, 1).wait()


def kernel(embeddings, x):
    xt = x.T.reshape(-1).astype(jnp.int32)
    mesh = plsc.VectorSubcoreMesh(core_axis_name="c", subcore_axis_name="s")
    f = pl.kernel(
        _body,
        mesh=mesh,
        compiler_params=pltpu.CompilerParams(
            use_tc_tiling_on_sc=False, needs_layout_passes=False),
        out_type=jax.ShapeDtypeStruct((N * EMBED_DIM,), jnp.float32),
        scratch_types=[
            pltpu.VMEM((2, UNIT), jnp.int32),
            pltpu.VMEM((2, UNIT, EMBED_DIM), jnp.float32),
            pltpu.VMEM((2, UNIT * EMBED_DIM), jnp.float32),
            pltpu.SemaphoreType.DMA,
            pltpu.SemaphoreType.DMA,
            pltpu.SemaphoreType.DMA,
        ],
    )
    out_flat = f(embeddings, xt)
    out5 = out_flat.reshape(HIST, 2, 32, 8, 128)
    return out5.transpose(2, 4, 0, 1, 3).reshape(BATCH, HIST, EMBED_DIM)


# speedup vs baseline: 1.5128x; 1.1065x over previous
import jax
import jax.numpy as jnp
from jax import lax
from jax.experimental import pallas as pl
from jax.experimental.pallas import tpu as pltpu
from jax.experimental.pallas import tpu_sc as plsc

BATCH = 4096
HIST = 200
EMBED_DIM = 16
N = BATCH * HIST
NUM_CORES = 2
NUM_SUBCORES = 16
NW = NUM_CORES * NUM_SUBCORES
UNIT = 1024
NUNIT = N // UNIT
PER_W = NUNIT // NW


def _body(tab_hbm, idx_hbm, out_hbm, idx_vs, g_bufs, w_bufs,
          sem_i, sem_g, sem_w):
    wid = lax.axis_index("s") * NUM_CORES + lax.axis_index("c")
    lane = lax.iota(jnp.int32, 16)
    pattern = (lane >> 3) * 8192 + (lane & 7) * 128

    def unit_off(t):
        u = wid + t * NW
        h = u >> 2
        q = u & 3
        return h * BATCH + q * UNIT, h * 65536 + q * 8192

    def idx_copy(t):
        src_off, _ = unit_off(t)
        return pltpu.make_async_copy(
            idx_hbm.at[pl.ds(src_off, UNIT)], idx_vs.at[t & 1], sem_i)

    def gather_copy(t):
        par = t & 1
        return pltpu.make_async_copy(
            tab_hbm.at[idx_vs.at[par]], g_bufs.at[par], sem_g)

    def write_copy(t, p):
        _, dst_off = unit_off(t)
        return pltpu.make_async_copy(
            w_bufs.at[t & 1, pl.ds(p * 8192, 8192)],
            out_hbm.at[pl.ds(dst_off + p * 32768, 8192)], sem_w)

    idx_copy(0).start()
    idx_copy(0).wait()
    gather_copy(0).start()
    idx_copy(1).start()

    def step(t, carry):
        par = t & 1
        gather_copy(t).wait()

        @pl.when(t + 1 < PER_W)
        def _():
            idx_copy(t + 1).wait()
            gather_copy(t + 1).start()

        @pl.when(t + 2 < PER_W)
        def _():
            idx_copy(t + 2).start()

        @pl.when(t >= 2)
        def _():
            write_copy(t - 2, 0).wait()
            write_copy(t - 2, 1).wait()

        @plsc.parallel_loop(0, UNIT, unroll=8)
        def _trans(i):
            row = g_bufs[par, i]
            base = ((i >> 7) << 10) + (i & 127)
            plsc.store_scatter(w_bufs.at[par], [pattern + base], row)
        write_copy(t, 0).start()
        write_copy(t, 1).start()
        return carry

    lax.fori_loop(0, PER_W, step, 0)
    write_copy(PER_W - 2, 0).wait()
    write_copy(PER_W - 2, 1).wait()
    write_copy(PER_W - 1, 0).wait()
    write_copy(PER_W - 1, 1).wait()


def kernel(embeddings, x):
    xt = x.T.reshape(-1).astype(jnp.int32)
    mesh = plsc.VectorSubcoreMesh(core_axis_name="c", subcore_axis_name="s")
    f = pl.kernel(
        _body,
        mesh=mesh,
        compiler_params=pltpu.CompilerParams(
            use_tc_tiling_on_sc=False, needs_layout_passes=False),
        out_type=jax.ShapeDtypeStruct((N * EMBED_DIM,), jnp.float32),
        scratch_types=[
            pltpu.VMEM((2, UNIT), jnp.int32),
            pltpu.VMEM((2, UNIT, EMBED_DIM), jnp.float32),
            pltpu.VMEM((2, UNIT * EMBED_DIM), jnp.float32),
            pltpu.SemaphoreType.DMA,
            pltpu.SemaphoreType.DMA,
            pltpu.SemaphoreType.DMA,
        ],
    )
    out_flat = f(embeddings, xt)
    out5 = out_flat.reshape(HIST, 2, 32, 8, 128)
    return out5.transpose(2, 4, 0, 1, 3).reshape(BATCH, HIST, EMBED_DIM)


# speedup vs baseline: 2.2929x; 1.5156x over previous
import jax
import jax.numpy as jnp
from jax import lax
from jax.experimental import pallas as pl
from jax.experimental.pallas import tpu as pltpu
from jax.experimental.pallas import tpu_sc as plsc

BATCH = 4096
HIST = 200
EMBED_DIM = 16
N = BATCH * HIST
VOCAB_ROWS = 1000000
NUM_CORES = 2
NUM_SUBCORES = 16
NW = NUM_CORES * NUM_SUBCORES

VTILES = VOCAB_ROWS // 128
TAIL = VOCAB_ROWS - VTILES * 128
CONV_PER_W = VTILES // NW + 1

UNIT = 1024
NUNIT = N // UNIT
PER_W = NUNIT // NW


def _conv_body(tabt_hbm, rm_hbm, in_bufs, w_bufs, tail_in, tail_w,
               sem_i, sem_o):
    wid = lax.axis_index("s") * NUM_CORES + lax.axis_index("c")
    lane = lax.iota(jnp.int32, 16)

    def in_copy(t):
        vt = wid + t * NW
        return pltpu.make_async_copy(
            tabt_hbm.at[:, pl.ds(vt * 128, 128)], in_bufs.at[t & 1], sem_i)

    def out_copy(t):
        vt = wid + t * NW
        return pltpu.make_async_copy(
            w_bufs.at[t & 1], rm_hbm.at[pl.ds(vt * 2048, 2048)], sem_o)

    in_copy(0).start()

    def step(t, carry):
        valid = wid + t * NW < VTILES

        @pl.when(valid)
        def _():
            par = t & 1
            in_copy(t).wait()

            @pl.when(wid + (t + 1) * NW < VTILES)
            def _():
                in_copy(t + 1).start()

            @pl.when(t >= 2)
            def _():
                out_copy(t - 2).wait()

            @plsc.parallel_loop(0, 128, unroll=8)
            def _trans(c):
                v = plsc.load_gather(in_bufs.at[par], [lane, lane * 0 + c])
                w_bufs[par, pl.ds(c * 16, 16)] = v

            out_copy(t).start()

        return carry

    lax.fori_loop(0, CONV_PER_W, step, 0)
    out_copy(0).wait()
    out_copy(1).wait()

    @pl.when(wid == 0)
    def _():
        pltpu.sync_copy(tabt_hbm.at[:, pl.ds(VTILES * 128, TAIL)], tail_in)

        @plsc.parallel_loop(0, TAIL, unroll=8)
        def _trans(c):
            v = plsc.load_gather(tail_in, [lane, lane * 0 + c])
            tail_w[pl.ds(c * 16, 16)] = v

        pltpu.sync_copy(
            tail_w, rm_hbm.at[pl.ds(VTILES * 2048, TAIL * EMBED_DIM)])


def _gather_body(tab_hbm, idx_hbm, out_hbm, idx_vs, g_bufs, w_bufs,
                 sem_i, sem_g, sem_w):
    wid = lax.axis_index("s") * NUM_CORES + lax.axis_index("c")
    lane = lax.iota(jnp.int32, 16)
    pattern = (lane >> 3) * 8192 + (lane & 7) * 128

    def unit_off(t):
        u = wid + t * NW
        h = u >> 2
        q = u & 3
        return h * BATCH + q * UNIT, h * 65536 + q * 8192

    def idx_copy(t):
        src_off, _ = unit_off(t)
        return pltpu.make_async_copy(
            idx_hbm.at[pl.ds(src_off, UNIT)], idx_vs.at[t & 1], sem_i)

    def gather_copy(t):
        par = t & 1
        return pltpu.make_async_copy(
            tab_hbm.at[idx_vs.at[par]], g_bufs.at[par], sem_g)

    def write_copy(t, p):
        _, dst_off = unit_off(t)
        return pltpu.make_async_copy(
            w_bufs.at[t & 1, pl.ds(p * 8192, 8192)],
            out_hbm.at[pl.ds(dst_off + p * 32768, 8192)], sem_w)

    idx_copy(0).start()
    idx_copy(0).wait()
    gather_copy(0).start()
    idx_copy(1).start()

    def step(t, carry):
        par = t & 1
        gather_copy(t).wait()

        @pl.when(t + 1 < PER_W)
        def _():
            idx_copy(t + 1).wait()
            gather_copy(t + 1).start()

        @pl.when(t + 2 < PER_W)
        def _():
            idx_copy(t + 2).start()

        @pl.when(t >= 2)
        def _():
            write_copy(t - 2, 0).wait()
            write_copy(t - 2, 1).wait()

        @plsc.parallel_loop(0, UNIT, unroll=8)
        def _trans(i):
            row = g_bufs[par, i]
            base = ((i >> 7) << 10) + (i & 127)
            plsc.store_scatter(w_bufs.at[par], [pattern + base], row)

        write_copy(t, 0).start()
        write_copy(t, 1).start()
        return carry

    lax.fori_loop(0, PER_W, step, 0)
    write_copy(PER_W - 2, 0).wait()
    write_copy(PER_W - 2, 1).wait()
    write_copy(PER_W - 1, 0).wait()
    write_copy(PER_W - 1, 1).wait()


def kernel(embeddings, x):
    mesh = plsc.VectorSubcoreMesh(core_axis_name="c", subcore_axis_name="s")

    conv = pl.kernel(
        _conv_body,
        mesh=mesh,
        compiler_params=pltpu.CompilerParams(
            use_tc_tiling_on_sc=True, needs_layout_passes=False),
        out_type=jax.ShapeDtypeStruct((VOCAB_ROWS * EMBED_DIM,), jnp.float32),
        scratch_types=[
            pltpu.VMEM((2, EMBED_DIM, 128), jnp.float32),
            pltpu.VMEM((2, 2048), jnp.float32),
            pltpu.VMEM((EMBED_DIM, TAIL), jnp.float32),
            pltpu.VMEM((TAIL * EMBED_DIM,), jnp.float32),
            pltpu.SemaphoreType.DMA,
            pltpu.SemaphoreType.DMA,
        ],
    )
    rm_flat = conv(embeddings.T)
    rm = rm_flat.reshape(VOCAB_ROWS, EMBED_DIM)

    xt = x.T.reshape(-1).astype(jnp.int32)
    gather = pl.kernel(
        _gather_body,
        mesh=mesh,
        compiler_params=pltpu.CompilerParams(
            use_tc_tiling_on_sc=False, needs_layout_passes=False),
        out_type=jax.ShapeDtypeStruct((N * EMBED_DIM,), jnp.float32),
        scratch_types=[
            pltpu.VMEM((2, UNIT), jnp.int32),
            pltpu.VMEM((2, UNIT, EMBED_DIM), jnp.float32),
            pltpu.VMEM((2, UNIT * EMBED_DIM), jnp.float32),
            pltpu.SemaphoreType.DMA,
            pltpu.SemaphoreType.DMA,
            pltpu.SemaphoreType.DMA,
        ],
    )
    out_flat = gather(rm, xt)
    out5 = out_flat.reshape(HIST, 2, 32, 8, 128)
    return out5.transpose(2, 4, 0, 1, 3).reshape(BATCH, HIST, EMBED_DIM)


# speedup vs baseline: 3.6707x; 1.6009x over previous
import jax
import jax.numpy as jnp
from jax import lax
from jax.experimental import pallas as pl
from jax.experimental.pallas import tpu as pltpu
from jax.experimental.pallas import tpu_sc as plsc

BATCH = 4096
HIST = 200
EMBED_DIM = 16
N = BATCH * HIST
VOCAB_ROWS = 1000000
NUM_CORES = 2
NUM_SUBCORES = 16
NW = NUM_CORES * NUM_SUBCORES

VTILES = VOCAB_ROWS // 128
TAIL = VOCAB_ROWS - VTILES * 128
CONV_PER_W = VTILES // NW + 1

UNIT = 1024
NUNIT = N // UNIT
PER_W = NUNIT // NW


def _conv_body(tabt_hbm, aux_hbm, rm_hbm, in_bufs, w_bufs, aux_v, tail_w,
               sem_i, sem_o):
    wid = lax.axis_index("s") * NUM_CORES + lax.axis_index("c")
    lane = lax.iota(jnp.int32, 16)

    def in_copy(t):
        vt = wid + t * NW
        return pltpu.make_async_copy(
            tabt_hbm.at[:, pl.ds(vt * 128, 128)], in_bufs.at[t & 1], sem_i)

    def out_copy(t):
        vt = wid + t * NW
        return pltpu.make_async_copy(
            w_bufs.at[t & 1], rm_hbm.at[pl.ds(vt * 2048, 2048)], sem_o)

    in_copy(0).start()

    def step(t, carry):
        valid = wid + t * NW < VTILES

        @pl.when(valid)
        def _():
            par = t & 1
            in_copy(t).wait()

            @pl.when(wid + (t + 1) * NW < VTILES)
            def _():
                in_copy(t + 1).start()

            @pl.when(t >= 2)
            def _():
                out_copy(t - 2).wait()

            @plsc.parallel_loop(0, 128, unroll=8)
            def _trans(m):
                c_vec = (m & ~15) + ((lane + m) & 15)
                v = plsc.load_gather(in_bufs.at[par], [lane, c_vec])
                plsc.store_scatter(
                    w_bufs, [lane * 0 + par, c_vec * 16 + lane], v)

            out_copy(t).start()

        return carry

    lax.fori_loop(0, CONV_PER_W, step, 0)
    out_copy(0).wait()
    out_copy(1).wait()

    @pl.when(wid == 0)
    def _():
        pltpu.sync_copy(aux_hbm, aux_v)

        @plsc.parallel_loop(0, TAIL, unroll=8)
        def _row(r):
            v = plsc.load_gather(aux_v, [lane * 0 + r, lane])
            tail_w[pl.ds(r * 16, 16)] = v

        pltpu.sync_copy(
            tail_w, rm_hbm.at[pl.ds(VTILES * 2048, TAIL * EMBED_DIM)])


def _gather_body(tab_hbm, idx_hbm, out_hbm, idx_vs, g_bufs, w_bufs,
                 sem_i, sem_g, sem_w):
    wid = lax.axis_index("s") * NUM_CORES + lax.axis_index("c")
    lane = lax.iota(jnp.int32, 16)
    lane_part = (lane >> 3) * 8192 + (lane & 7) * 128

    def unit_off(t):
        u = wid + t * NW
        h = u >> 2
        q = u & 3
        return h * BATCH + q * UNIT, h * 65536 + q * 8192

    def idx_copy(t):
        src_off, _ = unit_off(t)
        return pltpu.make_async_copy(
            idx_hbm.at[pl.ds(src_off, UNIT)], idx_vs.at[t & 1], sem_i)

    def gather_copy(t):
        par = t & 1
        return pltpu.make_async_copy(
            tab_hbm.at[idx_vs.at[par]], g_bufs.at[par], sem_g)

    def write_copy(t, p):
        _, dst_off = unit_off(t)
        return pltpu.make_async_copy(
            w_bufs.at[t & 1, pl.ds(p * 8192, 8192)],
            out_hbm.at[pl.ds(dst_off + p * 32768, 8192)], sem_w)

    idx_copy(0).start()
    idx_copy(0).wait()
    gather_copy(0).start()
    idx_copy(1).start()

    def step(t, carry):
        par = t & 1
        gather_copy(t).wait()

        @pl.when(t + 1 < PER_W)
        def _():
            idx_copy(t + 1).wait()
            gather_copy(t + 1).start()

        @pl.when(t + 2 < PER_W)
        def _():
            idx_copy(t + 2).start()

        @pl.when(t >= 2)
        def _():
            write_copy(t - 2, 0).wait()
            write_copy(t - 2, 1).wait()

        @plsc.parallel_loop(0, UNIT, unroll=8)
        def _trans(m):
            g0 = m >> 4
            i_vec = (m & ~15) + ((lane + m) & 15)
            v = plsc.load_gather(g_bufs.at[par], [i_vec, lane])
            base = ((g0 >> 3) << 10) + ((g0 & 7) << 4)
            addr = lane_part + base + ((lane + m) & 15)
            plsc.store_scatter(w_bufs.at[par], [addr], v)

        write_copy(t, 0).start()
        write_copy(t, 1).start()
        return carry

    lax.fori_loop(0, PER_W, step, 0)
    write_copy(PER_W - 2, 0).wait()
    write_copy(PER_W - 2, 1).wait()
    write_copy(PER_W - 1, 0).wait()
    write_copy(PER_W - 1, 1).wait()


def kernel(embeddings, x):
    mesh = plsc.VectorSubcoreMesh(core_axis_name="c", subcore_axis_name="s")

    conv = pl.kernel(
        _conv_body,
        mesh=mesh,
        compiler_params=pltpu.CompilerParams(
            use_tc_tiling_on_sc=True, needs_layout_passes=False),
        out_type=jax.ShapeDtypeStruct((VOCAB_ROWS * EMBED_DIM,), jnp.float32),
        scratch_types=[
            pltpu.VMEM((2, EMBED_DIM, 128), jnp.float32),
            pltpu.VMEM((2, 2048), jnp.float32),
            pltpu.VMEM((TAIL, EMBED_DIM), jnp.float32),
            pltpu.VMEM((TAIL * EMBED_DIM,), jnp.float32),
            pltpu.SemaphoreType.DMA,
            pltpu.SemaphoreType.DMA,
        ],
    )
    rm_flat = conv(embeddings.T, embeddings[VTILES * 128:])
    rm = rm_flat.reshape(VOCAB_ROWS, EMBED_DIM)

    xt = x.T.reshape(-1).astype(jnp.int32)
    gather = pl.kernel(
        _gather_body,
        mesh=mesh,
        compiler_params=pltpu.CompilerParams(
            use_tc_tiling_on_sc=False, needs_layout_passes=False),
        out_type=jax.ShapeDtypeStruct((N * EMBED_DIM,), jnp.float32),
        scratch_types=[
            pltpu.VMEM((2, UNIT), jnp.int32),
            pltpu.VMEM((2, UNIT, EMBED_DIM), jnp.float32),
            pltpu.VMEM((2, UNIT * EMBED_DIM), jnp.float32),
            pltpu.SemaphoreType.DMA,
            pltpu.SemaphoreType.DMA,
            pltpu.SemaphoreType.DMA,
        ],
    )
    out_flat = gather(rm, xt)
    out5 = out_flat.reshape(HIST, 2, 32, 8, 128)
    return out5.transpose(2, 4, 0, 1, 3).reshape(BATCH, HIST, EMBED_DIM)


# speedup vs baseline: 6.1201x; 1.6673x over previous
import jax
import jax.numpy as jnp
from jax import lax
from jax.experimental import pallas as pl
from jax.experimental.pallas import tpu as pltpu
from jax.experimental.pallas import tpu_sc as plsc

BATCH = 4096
HIST = 200
EMBED_DIM = 16
N = BATCH * HIST
VOCAB_ROWS = 1000000
NUM_CORES = 2
NUM_SUBCORES = 16
NW = NUM_CORES * NUM_SUBCORES

VTILES = VOCAB_ROWS // 128
TAIL = VOCAB_ROWS - VTILES * 128
CGRP = 4
CBLK = 128 * CGRP
NCBLK = VTILES // CGRP
CONV_PER_W = NCBLK // NW + 1

UNIT = 1024
NUNIT = N // UNIT
PER_W = NUNIT // NW


def _conv_body(tabt_hbm, aux_hbm, rm_hbm, in_bufs, w_bufs, aux_v, tail_w,
               sem_i, sem_o):
    wid = lax.axis_index("s") * NUM_CORES + lax.axis_index("c")
    lane = lax.iota(jnp.int32, 16)
    perms = [(lane + k) & 15 for k in range(16)]
    scats = [p * 16 + lane for p in perms]

    def in_copy(t):
        blk = wid + t * NW
        return pltpu.make_async_copy(
            tabt_hbm.at[:, pl.ds(blk * CBLK, CBLK)], in_bufs.at[t & 1], sem_i)

    def out_copy(t):
        blk = wid + t * NW
        return pltpu.make_async_copy(
            w_bufs.at[t & 1],
            rm_hbm.at[pl.ds(blk * CBLK * EMBED_DIM, CBLK * EMBED_DIM)], sem_o)

    in_copy(0).start()

    def step(t, carry):
        valid = wid + t * NW < NCBLK

        @pl.when(valid)
        def _():
            par = t & 1
            in_copy(t).wait()

            @pl.when(wid + (t + 1) * NW < NCBLK)
            def _():
                in_copy(t + 1).start()

            @pl.when(t >= 2)
            def _():
                out_copy(t - 2).wait()

            @plsc.parallel_loop(0, CBLK // 16, unroll=2)
            def _trans(g):
                c16 = g * 16
                s16 = g * 256
                for k in range(16):
                    v = plsc.load_gather(
                        in_bufs.at[par], [lane, perms[k] + c16])
                    plsc.store_scatter(
                        w_bufs, [lane * 0 + par, scats[k] + s16], v)

            out_copy(t).start()

        return carry

    lax.fori_loop(0, CONV_PER_W, step, 0)
    out_copy(0).wait()
    out_copy(1).wait()

    @pl.when(wid == 0)
    def _():
        pltpu.sync_copy(aux_hbm, aux_v)

        @plsc.parallel_loop(0, TAIL, unroll=8)
        def _row(r):
            v = plsc.load_gather(aux_v, [lane * 0 + r, lane])
            tail_w[pl.ds(r * 16, 16)] = v

        pltpu.sync_copy(
            tail_w, rm_hbm.at[pl.ds(VTILES * 2048, TAIL * EMBED_DIM)])


def _gather_body(tab_hbm, idx_hbm, out_hbm, idx_vs, g_bufs, w_bufs,
                 sem_i, sem_g, sem_w):
    wid = lax.axis_index("s") * NUM_CORES + lax.axis_index("c")
    lane = lax.iota(jnp.int32, 16)
    lane_part = (lane >> 3) * 8192 + (lane & 7) * 128

    def unit_off(t):
        u = wid + t * NW
        h = u >> 2
        q = u & 3
        return h * BATCH + q * UNIT, h * 65536 + q * 8192

    def idx_copy(t):
        src_off, _ = unit_off(t)
        return pltpu.make_async_copy(
            idx_hbm.at[pl.ds(src_off, UNIT)], idx_vs.at[t & 1], sem_i)

    def gather_copy(t):
        par = t & 1
        return pltpu.make_async_copy(
            tab_hbm.at[idx_vs.at[par]], g_bufs.at[par], sem_g)

    def write_copy(t, p):
        _, dst_off = unit_off(t)
        return pltpu.make_async_copy(
            w_bufs.at[t & 1, pl.ds(p * 8192, 8192)],
            out_hbm.at[pl.ds(dst_off + p * 32768, 8192)], sem_w)

    idx_copy(0).start()
    idx_copy(0).wait()
    gather_copy(0).start()
    idx_copy(1).start()

    def step(t, carry):
        par = t & 1
        gather_copy(t).wait()

        @pl.when(t + 1 < PER_W)
        def _():
            idx_copy(t + 1).wait()
            gather_copy(t + 1).start()

        @pl.when(t + 2 < PER_W)
        def _():
            idx_copy(t + 2).start()

        @pl.when(t >= 2)
        def _():
            write_copy(t - 2, 0).wait()
            write_copy(t - 2, 1).wait()

        @plsc.parallel_loop(0, UNIT, unroll=8)
        def _trans(m):
            g0 = m >> 4
            i_vec = (m & ~15) + ((lane + m) & 15)
            v = plsc.load_gather(g_bufs.at[par], [i_vec, lane])
            base = ((g0 >> 3) << 10) + ((g0 & 7) << 4)
            addr = lane_part + base + ((lane + m) & 15)
            plsc.store_scatter(w_bufs.at[par], [addr], v)

        write_copy(t, 0).start()
        write_copy(t, 1).start()
        return carry

    lax.fori_loop(0, PER_W, step, 0)
    write_copy(PER_W - 2, 0).wait()
    write_copy(PER_W - 2, 1).wait()
    write_copy(PER_W - 1, 0).wait()
    write_copy(PER_W - 1, 1).wait()


def kernel(embeddings, x):
    mesh = plsc.VectorSubcoreMesh(core_axis_name="c", subcore_axis_name="s")

    conv = pl.kernel(
        _conv_body,
        mesh=mesh,
        compiler_params=pltpu.CompilerParams(
            use_tc_tiling_on_sc=True, needs_layout_passes=False),
        out_type=jax.ShapeDtypeStruct((VOCAB_ROWS * EMBED_DIM,), jnp.float32),
        scratch_types=[
            pltpu.VMEM((2, EMBED_DIM, CBLK), jnp.float32),
            pltpu.VMEM((2, CBLK * EMBED_DIM), jnp.float32),
            pltpu.VMEM((TAIL, EMBED_DIM), jnp.float32),
            pltpu.VMEM((TAIL * EMBED_DIM,), jnp.float32),
            pltpu.SemaphoreType.DMA,
            pltpu.SemaphoreType.DMA,
        ],
    )
    rm_flat = conv(embeddings.T, embeddings[VTILES * 128:])
    rm = rm_flat.reshape(VOCAB_ROWS, EMBED_DIM)

    xt = x.T.reshape(-1).astype(jnp.int32)
    gather = pl.kernel(
        _gather_body,
        mesh=mesh,
        compiler_params=pltpu.CompilerParams(
            use_tc_tiling_on_sc=False, needs_layout_passes=False),
        out_type=jax.ShapeDtypeStruct((N * EMBED_DIM,), jnp.float32),
        scratch_types=[
            pltpu.VMEM((2, UNIT), jnp.int32),
            pltpu.VMEM((2, UNIT, EMBED_DIM), jnp.float32),
            pltpu.VMEM((2, UNIT * EMBED_DIM), jnp.float32),
            pltpu.SemaphoreType.DMA,
            pltpu.SemaphoreType.DMA,
            pltpu.SemaphoreType.DMA,
        ],
    )
    out_flat = gather(rm, xt)
    out5 = out_flat.reshape(HIST, 2, 32, 8, 128)
    return out5.transpose(2, 4, 0, 1, 3).reshape(BATCH, HIST, EMBED_DIM)


# speedup vs baseline: 6.4790x; 1.0586x over previous
import jax
import jax.numpy as jnp
from jax import lax
from jax.experimental import pallas as pl
from jax.experimental.pallas import tpu as pltpu
from jax.experimental.pallas import tpu_sc as plsc

BATCH = 4096
HIST = 200
EMBED_DIM = 16
N = BATCH * HIST
VOCAB_ROWS = 1000000
NUM_CORES = 2
NUM_SUBCORES = 16
NW = NUM_CORES * NUM_SUBCORES

VTILES = VOCAB_ROWS // 128
TAIL = VOCAB_ROWS - VTILES * 128
CGRP = 6
CBLK = 128 * CGRP
NCBLK = VTILES // CGRP
CONV_PER_W = NCBLK // NW + 1

UNIT = 1024
NUNIT = N // UNIT
PER_W = NUNIT // NW


def _conv_body(tabt_hbm, aux_hbm, rm_hbm, in_bufs, w_bufs, aux_v, tail_w,
               sem_i, sem_o):
    wid = lax.axis_index("s") * NUM_CORES + lax.axis_index("c")
    lane = lax.iota(jnp.int32, 16)
    perms = [(lane + k) & 15 for k in range(16)]
    scats = [p * 16 + lane for p in perms]

    def in_copy(t):
        blk = wid + t * NW
        return pltpu.make_async_copy(
            tabt_hbm.at[:, pl.ds(blk * CBLK, CBLK)], in_bufs.at[t & 1], sem_i)

    def out_copy(t):
        blk = wid + t * NW
        return pltpu.make_async_copy(
            w_bufs.at[t & 1],
            rm_hbm.at[pl.ds(blk * CBLK * EMBED_DIM, CBLK * EMBED_DIM)], sem_o)

    in_copy(0).start()

    def step(t, carry):
        valid = wid + t * NW < NCBLK

        @pl.when(valid)
        def _():
            par = t & 1

            @pl.when(wid + (t + 1) * NW < NCBLK)
            def _():
                in_copy(t + 1).start()

            in_copy(t).wait()

            @pl.when(t >= 2)
            def _():
                out_copy(t - 2).wait()

            @plsc.parallel_loop(0, CBLK // 16, unroll=2)
            def _trans(g):
                c16 = g * 16
                s16 = g * 256
                for k in range(16):
                    v = plsc.load_gather(
                        in_bufs.at[par], [lane, perms[k] + c16])
                    plsc.store_scatter(
                        w_bufs, [lane * 0 + par, scats[k] + s16], v)

            out_copy(t).start()

        return carry

    lax.fori_loop(0, CONV_PER_W, step, 0)
    out_copy(0).wait()
    out_copy(1).wait()

    @pl.when(wid == 0)
    def _():
        pltpu.sync_copy(aux_hbm, aux_v)

        @plsc.parallel_loop(0, TAIL, unroll=8)
        def _row(r):
            v = plsc.load_gather(aux_v, [lane * 0 + r, lane])
            tail_w[pl.ds(r * 16, 16)] = v

        pltpu.sync_copy(
            tail_w, rm_hbm.at[pl.ds(VTILES * 2048, TAIL * EMBED_DIM)])


def _gather_body(tab_hbm, idx_hbm, out_hbm, idx_vs, g_bufs, w_bufs,
                 sem_i, sem_g, sem_w):
    wid = lax.axis_index("s") * NUM_CORES + lax.axis_index("c")
    lane = lax.iota(jnp.int32, 16)
    lane_part = (lane >> 3) * 8192 + (lane & 7) * 128

    def unit_off(t):
        u = wid + t * NW
        h = u >> 2
        q = u & 3
        return h * BATCH + q * UNIT, h * 65536 + q * 8192

    def idx_copy(t):
        src_off, _ = unit_off(t)
        return pltpu.make_async_copy(
            idx_hbm.at[pl.ds(src_off, UNIT)], idx_vs.at[t & 1], sem_i)

    def gather_copy(t):
        par = t & 1
        return pltpu.make_async_copy(
            tab_hbm.at[idx_vs.at[par]], g_bufs.at[par], sem_g)

    def write_copy(t, p):
        _, dst_off = unit_off(t)
        return pltpu.make_async_copy(
            w_bufs.at[t & 1, pl.ds(p * 8192, 8192)],
            out_hbm.at[pl.ds(dst_off + p * 32768, 8192)], sem_w)

    idx_copy(0).start()
    idx_copy(0).wait()
    gather_copy(0).start()
    idx_copy(1).start()

    def step(t, carry):
        par = t & 1

        @pl.when(t + 1 < PER_W)
        def _():
            idx_copy(t + 1).wait()
            gather_copy(t + 1).start()

        gather_copy(t).wait()

        @pl.when(t + 2 < PER_W)
        def _():
            idx_copy(t + 2).start()

        @pl.when(t >= 2)
        def _():
            write_copy(t - 2, 0).wait()
            write_copy(t - 2, 1).wait()

        @plsc.parallel_loop(0, UNIT, unroll=8)
        def _trans(m):
            g0 = m >> 4
            i_vec = (m & ~15) + ((lane + m) & 15)
            v = plsc.load_gather(g_bufs.at[par], [i_vec, lane])
            base = ((g0 >> 3) << 10) + ((g0 & 7) << 4)
            addr = lane_part + base + ((lane + m) & 15)
            plsc.store_scatter(w_bufs.at[par], [addr], v)

        write_copy(t, 0).start()
        write_copy(t, 1).start()
        return carry

    lax.fori_loop(0, PER_W, step, 0)
    write_copy(PER_W - 2, 0).wait()
    write_copy(PER_W - 2, 1).wait()
    write_copy(PER_W - 1, 0).wait()
    write_copy(PER_W - 1, 1).wait()


def kernel(embeddings, x):
    mesh = plsc.VectorSubcoreMesh(core_axis_name="c", subcore_axis_name="s")

    conv = pl.kernel(
        _conv_body,
        mesh=mesh,
        compiler_params=pltpu.CompilerParams(
            use_tc_tiling_on_sc=True, needs_layout_passes=False),
        out_type=jax.ShapeDtypeStruct((VOCAB_ROWS * EMBED_DIM,), jnp.float32),
        scratch_types=[
            pltpu.VMEM((2, EMBED_DIM, CBLK), jnp.float32),
            pltpu.VMEM((2, CBLK * EMBED_DIM), jnp.float32),
            pltpu.VMEM((TAIL, EMBED_DIM), jnp.float32),
            pltpu.VMEM((TAIL * EMBED_DIM,), jnp.float32),
            pltpu.SemaphoreType.DMA,
            pltpu.SemaphoreType.DMA,
        ],
    )
    rm_flat = conv(embeddings.T, embeddings[VTILES * 128:])
    rm = rm_flat.reshape(VOCAB_ROWS, EMBED_DIM)

    xt = x.T.reshape(-1).astype(jnp.int32)
    gather = pl.kernel(
        _gather_body,
        mesh=mesh,
        compiler_params=pltpu.CompilerParams(
            use_tc_tiling_on_sc=False, needs_layout_passes=False),
        out_type=jax.ShapeDtypeStruct((N * EMBED_DIM,), jnp.float32),
        scratch_types=[
            pltpu.VMEM((2, UNIT), jnp.int32),
            pltpu.VMEM((2, UNIT, EMBED_DIM), jnp.float32),
            pltpu.VMEM((2, UNIT * EMBED_DIM), jnp.float32),
            pltpu.SemaphoreType.DMA,
            pltpu.SemaphoreType.DMA,
            pltpu.SemaphoreType.DMA,
        ],
    )
    out_flat = gather(rm, xt)
    out5 = out_flat.reshape(HIST, 2, 32, 8, 128)
    return out5.transpose(2, 4, 0, 1, 3).reshape(BATCH, HIST, EMBED_DIM)


# speedup vs baseline: 6.4863x; 1.0011x over previous
import jax
import jax.numpy as jnp
from jax import lax
from jax.experimental import pallas as pl
from jax.experimental.pallas import tpu as pltpu
from jax.experimental.pallas import tpu_sc as plsc

BATCH = 4096
HIST = 200
EMBED_DIM = 16
N = BATCH * HIST
VOCAB_ROWS = 1000000
NUM_CORES = 2
NUM_SUBCORES = 16
NW = NUM_CORES * NUM_SUBCORES

VTILES = VOCAB_ROWS // 128
TAIL = VOCAB_ROWS - VTILES * 128
CGRP = 12
CBLK = 128 * CGRP
NCBLK = VTILES // CGRP
CONV_PER_W = NCBLK // NW + 1

UNIT = 1024
NUNIT = N // UNIT
PER_W = NUNIT // NW


def _conv_body(tabt_hbm, aux_hbm, rm_hbm, in_bufs, w_bufs, aux_v, tail_w,
               sem_i, sem_o):
    wid = lax.axis_index("s") * NUM_CORES + lax.axis_index("c")
    lane = lax.iota(jnp.int32, 16)
    perms = [(lane + k) & 15 for k in range(16)]
    scats = [p * 16 + lane for p in perms]

    def in_copy(t):
        blk = wid + t * NW
        return pltpu.make_async_copy(
            tabt_hbm.at[:, pl.ds(blk * CBLK, CBLK)], in_bufs.at[t & 1], sem_i)

    def out_copy(t):
        blk = wid + t * NW
        return pltpu.make_async_copy(
            w_bufs.at[t & 1],
            rm_hbm.at[pl.ds(blk * CBLK * EMBED_DIM, CBLK * EMBED_DIM)], sem_o)

    in_copy(0).start()

    def step(t, carry):
        valid = wid + t * NW < NCBLK

        @pl.when(valid)
        def _():
            par = t & 1

            @pl.when(wid + (t + 1) * NW < NCBLK)
            def _():
                in_copy(t + 1).start()

            in_copy(t).wait()

            @pl.when(t >= 2)
            def _():
                out_copy(t - 2).wait()

            @plsc.parallel_loop(0, CBLK // 16, unroll=2)
            def _trans(g):
                c16 = g * 16
                s16 = g * 256
                for k in range(16):
                    v = plsc.load_gather(
                        in_bufs.at[par], [lane, perms[k] + c16])
                    plsc.store_scatter(
                        w_bufs, [lane * 0 + par, scats[k] + s16], v)

            out_copy(t).start()

        return carry

    lax.fori_loop(0, CONV_PER_W, step, 0)
    out_copy(0).wait()
    out_copy(1).wait()

    @pl.when(wid == 0)
    def _():
        pltpu.sync_copy(aux_hbm, aux_v)

        @plsc.parallel_loop(0, TAIL, unroll=8)
        def _row(r):
            v = plsc.load_gather(aux_v, [lane * 0 + r, lane])
            tail_w[pl.ds(r * 16, 16)] = v

        pltpu.sync_copy(
            tail_w, rm_hbm.at[pl.ds(VTILES * 2048, TAIL * EMBED_DIM)])


def _gather_body(tab_hbm, idx_hbm, out_hbm, idx_vs, g_bufs, w_bufs,
                 sem_i, sem_g, sem_w):
    wid = lax.axis_index("s") * NUM_CORES + lax.axis_index("c")
    lane = lax.iota(jnp.int32, 16)
    lane_part = (lane >> 3) * 8192 + (lane & 7) * 128

    def unit_off(t):
        u = wid + t * NW
        h = u >> 2
        q = u & 3
        return h * BATCH + q * UNIT, h * 65536 + q * 8192

    def idx_copy(t):
        src_off, _ = unit_off(t)
        return pltpu.make_async_copy(
            idx_hbm.at[pl.ds(src_off, UNIT)], idx_vs.at[t & 1], sem_i)

    def gather_copy(t):
        par = t & 1
        return pltpu.make_async_copy(
            tab_hbm.at[idx_vs.at[par]], g_bufs.at[par], sem_g)

    def write_copy(t, p):
        _, dst_off = unit_off(t)
        return pltpu.make_async_copy(
            w_bufs.at[t & 1, pl.ds(p * 8192, 8192)],
            out_hbm.at[pl.ds(dst_off + p * 32768, 8192)], sem_w)

    idx_copy(0).start()
    idx_copy(0).wait()
    gather_copy(0).start()
    idx_copy(1).start()

    def step(t, carry):
        par = t & 1

        @pl.when(t + 1 < PER_W)
        def _():
            idx_copy(t + 1).wait()
            gather_copy(t + 1).start()

        gather_copy(t).wait()

        @pl.when(t + 2 < PER_W)
        def _():
            idx_copy(t + 2).start()

        @pl.when(t >= 2)
        def _():
            write_copy(t - 2, 0).wait()
            write_copy(t - 2, 1).wait()

        @plsc.parallel_loop(0, UNIT, unroll=8)
        def _trans(m):
            g0 = m >> 4
            i_vec = (m & ~15) + ((lane + m) & 15)
            v = plsc.load_gather(g_bufs.at[par], [i_vec, lane])
            base = ((g0 >> 3) << 10) + ((g0 & 7) << 4)
            addr = lane_part + base + ((lane + m) & 15)
            plsc.store_scatter(w_bufs.at[par], [addr], v)

        write_copy(t, 0).start()
        write_copy(t, 1).start()
        return carry

    lax.fori_loop(0, PER_W, step, 0)
    write_copy(PER_W - 2, 0).wait()
    write_copy(PER_W - 2, 1).wait()
    write_copy(PER_W - 1, 0).wait()
    write_copy(PER_W - 1, 1).wait()


def kernel(embeddings, x):
    mesh = plsc.VectorSubcoreMesh(core_axis_name="c", subcore_axis_name="s")

    conv = pl.kernel(
        _conv_body,
        mesh=mesh,
        compiler_params=pltpu.CompilerParams(
            use_tc_tiling_on_sc=True, needs_layout_passes=False),
        out_type=jax.ShapeDtypeStruct((VOCAB_ROWS * EMBED_DIM,), jnp.float32),
        scratch_types=[
            pltpu.VMEM((2, EMBED_DIM, CBLK), jnp.float32),
            pltpu.VMEM((2, CBLK * EMBED_DIM), jnp.float32),
            pltpu.VMEM((TAIL, EMBED_DIM), jnp.float32),
            pltpu.VMEM((TAIL * EMBED_DIM,), jnp.float32),
            pltpu.SemaphoreType.DMA,
            pltpu.SemaphoreType.DMA,
        ],
    )
    rm_flat = conv(embeddings.T, embeddings[VTILES * 128:])
    rm = rm_flat.reshape(VOCAB_ROWS, EMBED_DIM)

    xt = x.T.reshape(-1).astype(jnp.int32)
    gather = pl.kernel(
        _gather_body,
        mesh=mesh,
        compiler_params=pltpu.CompilerParams(
            use_tc_tiling_on_sc=False, needs_layout_passes=False),
        out_type=jax.ShapeDtypeStruct((N * EMBED_DIM,), jnp.float32),
        scratch_types=[
            pltpu.VMEM((2, UNIT), jnp.int32),
            pltpu.VMEM((2, UNIT, EMBED_DIM), jnp.float32),
            pltpu.VMEM((2, UNIT * EMBED_DIM), jnp.float32),
            pltpu.SemaphoreType.DMA,
            pltpu.SemaphoreType.DMA,
            pltpu.SemaphoreType.DMA,
        ],
    )
    out_flat = gather(rm, xt)
    out5 = out_flat.reshape(HIST, 2, 32, 8, 128)
    return out5.transpose(2, 4, 0, 1, 3).reshape(BATCH, HIST, EMBED_DIM)
